# baseline (speedup 1.0000x reference)
import jax
import jax.numpy as jnp
from jax.experimental import pallas as pl
from jax.experimental.pallas import tpu as pltpu

N_TOKENS = 32768
HIDDEN = 768
N_EXPERTS = 64
TOP_K = 2
ROWS = N_TOKENS // 2
HID2 = 2 * HIDDEN
BLK = 1024
NBUF = 4
GRID = ROWS // BLK
NEG = float("-inf")


def _router_kernel(x_hbm, w2_ref, pk_ref, loss_ref, frac_ref, probs_ref,
                   xbuf, acc_ref, sems):
    i = pl.program_id(0)
    nsteps = pl.num_programs(0)

    @pl.when(i == 0)
    def _prologue():
        for b in range(NBUF):
            pltpu.make_async_copy(
                x_hbm.at[pl.ds(b * BLK, BLK), :], xbuf.at[b], sems.at[b]
            ).start()

    slot = jax.lax.rem(i, NBUF)
    pltpu.make_async_copy(
        x_hbm.at[pl.ds(i * BLK, BLK), :], xbuf.at[slot], sems.at[slot]
    ).wait()

    v = jax.lax.dot_general(
        xbuf[slot], w2_ref[...], (((1,), (0,)), ((), ())),
        preferred_element_type=jnp.float32)

    @pl.when(i + NBUF < nsteps)
    def _prefetch():
        pltpu.make_async_copy(
            x_hbm.at[pl.ds((i + NBUF) * BLK, BLK), :], xbuf.at[slot],
            sems.at[slot]
        ).start()

    cols = jax.lax.broadcasted_iota(jnp.int32, v.shape, 1)
    left = cols < N_EXPERTS
    ecols = jax.lax.bitwise_and(cols, N_EXPERTS - 1)

    va = jnp.where(left, v, NEG)
    vb = jnp.where(left, NEG, v)
    m1a = jnp.max(va, axis=1, keepdims=True)
    m1b = jnp.max(vb, axis=1, keepdims=True)
    i1a = jnp.min(jnp.where(va == m1a, ecols, N_EXPERTS), axis=1,
                  keepdims=True)
    i1b = jnp.min(jnp.where(vb == m1b, ecols, N_EXPERTS), axis=1,
                  keepdims=True)
    va2 = jnp.where(cols == i1a, NEG, va)
    vb2 = jnp.where(cols == i1b + N_EXPERTS, NEG, vb)
    m2a = jnp.max(va2, axis=1, keepdims=True)
    m2b = jnp.max(vb2, axis=1, keepdims=True)
    i2a = jnp.min(jnp.where(va2 == m2a, ecols, N_EXPERTS), axis=1,
                  keepdims=True)
    i2b = jnp.min(jnp.where(vb2 == m2b, ecols, N_EXPERTS), axis=1,
                  keepdims=True)

    m1 = jnp.where(left, m1a, m1b)
    e = jnp.exp(v - m1)
    sa = jnp.sum(jnp.where(left, e, 0.0), axis=1, keepdims=True)
    sb = jnp.sum(jnp.where(left, 0.0, e), axis=1, keepdims=True)
    probs = e / jnp.where(left, sa, sb)
    probs_ref[...] = probs

    w1a = 1.0 / (1.0 + jnp.exp(m2a - m1a))
    w1b = 1.0 / (1.0 + jnp.exp(m2b - m1b))

    s = jnp.concatenate(
        [w1a, 1.0 - w1a, i1a.astype(jnp.float32), i2a.astype(jnp.float32),
         w1b, 1.0 - w1b, i1b.astype(jnp.float32), i2b.astype(jnp.float32)],
        axis=1)
    pk_ref[...] = jnp.transpose(s)

    c1 = jnp.where(left, i1a, i1b + N_EXPERTS)
    c2 = jnp.where(left, i2a, i2b + N_EXPERTS)
    onehot = ((cols == c1).astype(jnp.float32)
              + (cols == c2).astype(jnp.float32))
    cnt = jnp.sum(onehot, axis=0, keepdims=True)
    ps = jnp.sum(probs, axis=0, keepdims=True)

    @pl.when(i == 0)
    def _init():
        acc_ref[...] = jnp.zeros_like(acc_ref)

    acc_ref[0:1] += cnt
    acc_ref[1:2] += ps

    @pl.when(i == nsteps - 1)
    def _fin():
        acc = acc_ref[...]
        counts = acc[0:1, :N_EXPERTS] + acc[0:1, N_EXPERTS:]
        tot = acc[1:2, :N_EXPERTS] + acc[1:2, N_EXPERTS:]
        inv_n = 1.0 / N_TOKENS
        loss_ref[...] = (N_EXPERTS * inv_n * inv_n) * jnp.sum(
            counts * tot, keepdims=True)
        frac_ref[...] = counts * inv_n


def kernel(x, W):
    x2 = x.reshape(ROWS, HID2)
    wt = W.T
    z = jnp.zeros_like(wt)
    w2 = jnp.concatenate(
        [jnp.concatenate([wt, z], axis=1),
         jnp.concatenate([z, wt], axis=1)], axis=0)

    pk, loss, frac, probs = pl.pallas_call(
        _router_kernel,
        grid=(GRID,),
        in_specs=[
            pl.BlockSpec(memory_space=pl.ANY),
            pl.BlockSpec((HID2, 2 * N_EXPERTS), lambda i: (0, 0)),
        ],
        out_specs=[
            pl.BlockSpec((8, BLK), lambda i: (0, i)),
            pl.BlockSpec((1, 1), lambda i: (0, 0)),
            pl.BlockSpec((1, N_EXPERTS), lambda i: (0, 0)),
            pl.BlockSpec((BLK, 2 * N_EXPERTS), lambda i: (i, 0)),
        ],
        out_shape=[
            jax.ShapeDtypeStruct((8, ROWS), jnp.float32),
            jax.ShapeDtypeStruct((1, 1), jnp.float32),
            jax.ShapeDtypeStruct((1, N_EXPERTS), jnp.float32),
            jax.ShapeDtypeStruct((ROWS, 2 * N_EXPERTS), jnp.float32),
        ],
        scratch_shapes=[
            pltpu.VMEM((NBUF, BLK, HID2), jnp.float32),
            pltpu.VMEM((2, 2 * N_EXPERTS), jnp.float32),
            pltpu.SemaphoreType.DMA((NBUF,)),
        ],
    )(x2, w2)

    def lace(a, b):
        return jnp.stack([a, b], axis=1).reshape(N_TOKENS)

    tw = jnp.stack([lace(pk[0], pk[4]), lace(pk[1], pk[5])], axis=1)
    idx = jnp.stack([lace(pk[2], pk[6]), lace(pk[3], pk[7])],
                    axis=1).astype(jnp.int32)
    return (idx, tw, loss[0, 0], frac[0],
            probs.reshape(N_TOKENS, N_EXPERTS))


# speedup vs baseline: 2.8805x; 2.8805x over previous
import jax
import jax.numpy as jnp
from jax.experimental import pallas as pl
from jax.experimental.pallas import tpu as pltpu

N_TOKENS = 32768
HIDDEN = 768
N_EXPERTS = 64
TOP_K = 2
BLK = 2048
NBUF = 4
GRID = N_TOKENS // BLK
NEG = float("-inf")


def _router_kernel(x_hbm, w_ref, idx_out, tw_out, loss_ref, frac_ref,
                   probs_out, xbuf, s_idx, s_tw, s_probs, psum_ref,
                   xsems, osems):
    i = pl.program_id(0)
    nsteps = pl.num_programs(0)

    @pl.when(i == 0)
    def _prologue():
        for b in range(NBUF):
            pltpu.make_async_copy(
                x_hbm.at[pl.ds(b * BLK, BLK), :], xbuf.at[b], xsems.at[b]
            ).start()

    slot = jax.lax.rem(i, NBUF)
    pltpu.make_async_copy(
        x_hbm.at[pl.ds(i * BLK, BLK), :], xbuf.at[slot], xsems.at[slot]
    ).wait()

    logits = jax.lax.dot_general(
        xbuf[slot], w_ref[...], (((1,), (1,)), ((), ())),
        preferred_element_type=jnp.float32)

    @pl.when(i + NBUF < nsteps)
    def _prefetch():
        pltpu.make_async_copy(
            x_hbm.at[pl.ds((i + NBUF) * BLK, BLK), :], xbuf.at[slot],
            xsems.at[slot]
        ).start()

    cols = jax.lax.broadcasted_iota(jnp.int32, logits.shape, 1)
    m1 = jnp.max(logits, axis=1, keepdims=True)
    idx1 = jnp.min(jnp.where(logits == m1, cols, N_EXPERTS), axis=1,
                   keepdims=True)
    masked = jnp.where(cols == idx1, NEG, logits)
    m2 = jnp.max(masked, axis=1, keepdims=True)
    idx2 = jnp.min(jnp.where(masked == m2, cols, N_EXPERTS), axis=1,
                   keepdims=True)

    e = jnp.exp(logits - m1)
    probs = e / jnp.sum(e, axis=1, keepdims=True)

    w1 = 1.0 / (1.0 + jnp.exp(m2 - m1))

    oslot = jax.lax.rem(i, 2)

    @pl.when(i >= 2)
    def _drain_prev():
        pltpu.make_async_copy(
            s_idx.at[oslot], idx_out.at[pl.ds((i - 2) * BLK, BLK), :],
            osems.at[0, oslot]).wait()
        pltpu.make_async_copy(
            s_tw.at[oslot], tw_out.at[pl.ds((i - 2) * BLK, BLK), :],
            osems.at[1, oslot]).wait()
        pltpu.make_async_copy(
            s_probs.at[oslot], probs_out.at[pl.ds((i - 2) * BLK, BLK), :],
            osems.at[2, oslot]).wait()

    s_idx[oslot] = jnp.concatenate([idx1, idx2], axis=1)
    s_tw[oslot] = jnp.concatenate([w1, 1.0 - w1], axis=1)
    s_probs[oslot] = probs

    pltpu.make_async_copy(
        s_idx.at[oslot], idx_out.at[pl.ds(i * BLK, BLK), :],
        osems.at[0, oslot]).start()
    pltpu.make_async_copy(
        s_tw.at[oslot], tw_out.at[pl.ds(i * BLK, BLK), :],
        osems.at[1, oslot]).start()
    pltpu.make_async_copy(
        s_probs.at[oslot], probs_out.at[pl.ds(i * BLK, BLK), :],
        osems.at[2, oslot]).start()

    onehot = ((cols == idx1).astype(jnp.float32)
              + (cols == idx2).astype(jnp.float32))
    cnt = jnp.sum(onehot, axis=0, keepdims=True)
    ps = jnp.sum(probs, axis=0, keepdims=True)

    @pl.when(i == 0)
    def _init():
        frac_ref[...] = jnp.zeros_like(frac_ref)
        psum_ref[...] = jnp.zeros_like(psum_ref)

    frac_ref[...] += cnt
    psum_ref[...] += ps

    @pl.when(i == nsteps - 1)
    def _fin():
        counts = frac_ref[...]
        inv_n = 1.0 / N_TOKENS
        loss_ref[...] = (N_EXPERTS * inv_n * inv_n) * jnp.sum(
            counts * psum_ref[...], keepdims=True)
        frac_ref[...] = counts * inv_n
        for step in (nsteps - 2, nsteps - 1):
            sl = step % 2
            pltpu.make_async_copy(
                s_idx.at[sl], idx_out.at[pl.ds(step * BLK, BLK), :],
                osems.at[0, sl]).wait()
            pltpu.make_async_copy(
                s_tw.at[sl], tw_out.at[pl.ds(step * BLK, BLK), :],
                osems.at[1, sl]).wait()
            pltpu.make_async_copy(
                s_probs.at[sl], probs_out.at[pl.ds(step * BLK, BLK), :],
                osems.at[2, sl]).wait()


def kernel(x, W):
    idx, tw, loss, frac, probs = pl.pallas_call(
        _router_kernel,
        grid=(GRID,),
        in_specs=[
            pl.BlockSpec(memory_space=pl.ANY),
            pl.BlockSpec((N_EXPERTS, HIDDEN), lambda i: (0, 0)),
        ],
        out_specs=[
            pl.BlockSpec(memory_space=pl.ANY),
            pl.BlockSpec(memory_space=pl.ANY),
            pl.BlockSpec((1, 1), lambda i: (0, 0)),
            pl.BlockSpec((1, N_EXPERTS), lambda i: (0, 0)),
            pl.BlockSpec(memory_space=pl.ANY),
        ],
        out_shape=[
            jax.ShapeDtypeStruct((N_TOKENS, TOP_K), jnp.int32),
            jax.ShapeDtypeStruct((N_TOKENS, TOP_K), jnp.float32),
            jax.ShapeDtypeStruct((1, 1), jnp.float32),
            jax.ShapeDtypeStruct((1, N_EXPERTS), jnp.float32),
            jax.ShapeDtypeStruct((N_TOKENS, N_EXPERTS), jnp.float32),
        ],
        scratch_shapes=[
            pltpu.VMEM((NBUF, BLK, HIDDEN), jnp.float32),
            pltpu.VMEM((2, BLK, TOP_K), jnp.int32),
            pltpu.VMEM((2, BLK, TOP_K), jnp.float32),
            pltpu.VMEM((2, BLK, N_EXPERTS), jnp.float32),
            pltpu.VMEM((1, N_EXPERTS), jnp.float32),
            pltpu.SemaphoreType.DMA((NBUF,)),
            pltpu.SemaphoreType.DMA((3, 2)),
        ],
    )(x, W)
    return idx, tw, loss[0, 0], frac[0], probs


# speedup vs baseline: 4.2330x; 1.4696x over previous
import jax
import jax.numpy as jnp
from jax.experimental import pallas as pl
from jax.experimental.pallas import tpu as pltpu

N_TOKENS = 32768
HIDDEN = 768
N_EXPERTS = 64
TOP_K = 2
BLK = 2048
NBUF = 4
GRID = N_TOKENS // BLK
NEG = float("-inf")


def _router_kernel(x_hbm, w_ref, idx_ref, tw_ref, loss_ref, frac_ref,
                   probs_ref, xbuf, psum_ref, xsems):
    i = pl.program_id(0)
    nsteps = pl.num_programs(0)

    @pl.when(i == 0)
    def _prologue():
        for b in range(NBUF):
            pltpu.make_async_copy(
                x_hbm.at[pl.ds(b * BLK, BLK), :], xbuf.at[b], xsems.at[b]
            ).start()

    slot = jax.lax.rem(i, NBUF)
    pltpu.make_async_copy(
        x_hbm.at[pl.ds(i * BLK, BLK), :], xbuf.at[slot], xsems.at[slot]
    ).wait()

    logits = jax.lax.dot_general(
        xbuf[slot], w_ref[...], (((1,), (1,)), ((), ())),
        preferred_element_type=jnp.float32)

    @pl.when(i + NBUF < nsteps)
    def _prefetch():
        pltpu.make_async_copy(
            x_hbm.at[pl.ds((i + NBUF) * BLK, BLK), :], xbuf.at[slot],
            xsems.at[slot]
        ).start()

    cols = jax.lax.broadcasted_iota(jnp.int32, logits.shape, 1)
    m1 = jnp.max(logits, axis=1, keepdims=True)
    idx1 = jnp.min(jnp.where(logits == m1, cols, N_EXPERTS), axis=1,
                   keepdims=True)
    masked = jnp.where(cols == idx1, NEG, logits)
    m2 = jnp.max(masked, axis=1, keepdims=True)
    idx2 = jnp.min(jnp.where(masked == m2, cols, N_EXPERTS), axis=1,
                   keepdims=True)

    e = jnp.exp(logits - m1)
    probs = e / jnp.sum(e, axis=1, keepdims=True)
    probs_ref[...] = jnp.transpose(probs)

    w1 = 1.0 / (1.0 + jnp.exp(m2 - m1))
    tw_ref[...] = jnp.transpose(jnp.concatenate([w1, 1.0 - w1], axis=1))
    idx_ref[...] = jnp.transpose(jnp.concatenate([idx1, idx2], axis=1))

    onehot = ((cols == idx1).astype(jnp.float32)
              + (cols == idx2).astype(jnp.float32))
    cnt = jnp.sum(onehot, axis=0, keepdims=True)
    ps = jnp.sum(probs, axis=0, keepdims=True)

    @pl.when(i == 0)
    def _init():
        frac_ref[...] = jnp.zeros_like(frac_ref)
        psum_ref[...] = jnp.zeros_like(psum_ref)

    frac_ref[...] += cnt
    psum_ref[...] += ps

    @pl.when(i == nsteps - 1)
    def _fin():
        counts = frac_ref[...]
        inv_n = 1.0 / N_TOKENS
        loss_ref[...] = (N_EXPERTS * inv_n * inv_n) * jnp.sum(
            counts * psum_ref[...], keepdims=True)
        frac_ref[...] = counts * inv_n


def kernel(x, W):
    idx_t, tw_t, loss, frac, probs_t = pl.pallas_call(
        _router_kernel,
        grid=(GRID,),
        in_specs=[
            pl.BlockSpec(memory_space=pl.ANY),
            pl.BlockSpec((N_EXPERTS, HIDDEN), lambda i: (0, 0)),
        ],
        out_specs=[
            pl.BlockSpec((TOP_K, BLK), lambda i: (0, i)),
            pl.BlockSpec((TOP_K, BLK), lambda i: (0, i)),
            pl.BlockSpec((1, 1), lambda i: (0, 0)),
            pl.BlockSpec((1, N_EXPERTS), lambda i: (0, 0)),
            pl.BlockSpec((N_EXPERTS, BLK), lambda i: (0, i)),
        ],
        out_shape=[
            jax.ShapeDtypeStruct((TOP_K, N_TOKENS), jnp.int32),
            jax.ShapeDtypeStruct((TOP_K, N_TOKENS), jnp.float32),
            jax.ShapeDtypeStruct((1, 1), jnp.float32),
            jax.ShapeDtypeStruct((1, N_EXPERTS), jnp.float32),
            jax.ShapeDtypeStruct((N_EXPERTS, N_TOKENS), jnp.float32),
        ],
        scratch_shapes=[
            pltpu.VMEM((NBUF, BLK, HIDDEN), jnp.float32),
            pltpu.VMEM((1, N_EXPERTS), jnp.float32),
            pltpu.SemaphoreType.DMA((NBUF,)),
        ],
    )(x, W)
    return (jnp.transpose(idx_t), jnp.transpose(tw_t), loss[0, 0],
            frac[0], jnp.transpose(probs_t))


# speedup vs baseline: 7.0472x; 1.6648x over previous
import jax
import jax.numpy as jnp
from jax.experimental import pallas as pl
from jax.experimental.pallas import tpu as pltpu

N_TOKENS = 32768
HIDDEN = 768
N_EXPERTS = 64
TOP_K = 2
BLK = 2048
NBUF = 4
GRID = N_TOKENS // BLK
NEG = float("-inf")


def _router_kernel(x_hbm, w_ref, idx_ref, tw_ref, loss_ref, frac_ref,
                   probs_ref, xbuf, acc_ref, xsems):
    i = pl.program_id(0)
    nsteps = pl.num_programs(0)

    @pl.when(i == 0)
    def _prologue():
        for b in range(NBUF):
            pltpu.make_async_copy(
                x_hbm.at[pl.ds(b * BLK, BLK), :], xbuf.at[b], xsems.at[b]
            ).start()

    slot = jax.lax.rem(i, NBUF)
    pltpu.make_async_copy(
        x_hbm.at[pl.ds(i * BLK, BLK), :], xbuf.at[slot], xsems.at[slot]
    ).wait()

    v = jax.lax.dot_general(
        w_ref[...], xbuf[slot], (((1,), (1,)), ((), ())),
        preferred_element_type=jnp.float32)

    @pl.when(i + NBUF < nsteps)
    def _prefetch():
        pltpu.make_async_copy(
            x_hbm.at[pl.ds((i + NBUF) * BLK, BLK), :], xbuf.at[slot],
            xsems.at[slot]
        ).start()

    rows = jax.lax.broadcasted_iota(jnp.int32, v.shape, 0)
    m1 = jnp.max(v, axis=0, keepdims=True)
    idx1 = jnp.min(jnp.where(v == m1, rows, N_EXPERTS), axis=0,
                   keepdims=True)
    masked = jnp.where(rows == idx1, NEG, v)
    m2 = jnp.max(masked, axis=0, keepdims=True)
    idx2 = jnp.min(jnp.where(masked == m2, rows, N_EXPERTS), axis=0,
                   keepdims=True)

    e = jnp.exp(v - m1)
    probs = e / jnp.sum(e, axis=0, keepdims=True)
    probs_ref[...] = probs

    w1 = 1.0 / (1.0 + jnp.exp(m2 - m1))
    tw_ref[...] = jnp.concatenate([w1, 1.0 - w1], axis=0)
    idx_ref[...] = jnp.concatenate([idx1, idx2], axis=0)

    onehot = ((rows == idx1).astype(jnp.float32)
              + (rows == idx2).astype(jnp.float32))
    cnt = jnp.sum(onehot, axis=1, keepdims=True)
    ps = jnp.sum(probs, axis=1, keepdims=True)

    @pl.when(i == 0)
    def _init():
        acc_ref[...] = jnp.zeros_like(acc_ref)

    acc_ref[:, 0:1] += cnt
    acc_ref[:, 1:2] += ps

    @pl.when(i == nsteps - 1)
    def _fin():
        counts = acc_ref[:, 0:1]
        tot = acc_ref[:, 1:2]
        inv_n = 1.0 / N_TOKENS
        loss_ref[...] = (N_EXPERTS * inv_n * inv_n) * jnp.sum(
            counts * tot, keepdims=True)
        frac_ref[...] = jnp.transpose(counts) * inv_n


def kernel(x, W):
    idx_t, tw_t, loss, frac, probs_t = pl.pallas_call(
        _router_kernel,
        grid=(GRID,),
        in_specs=[
            pl.BlockSpec(memory_space=pl.ANY),
            pl.BlockSpec((N_EXPERTS, HIDDEN), lambda i: (0, 0)),
        ],
        out_specs=[
            pl.BlockSpec((TOP_K, BLK), lambda i: (0, i)),
            pl.BlockSpec((TOP_K, BLK), lambda i: (0, i)),
            pl.BlockSpec((1, 1), lambda i: (0, 0)),
            pl.BlockSpec((1, N_EXPERTS), lambda i: (0, 0)),
            pl.BlockSpec((N_EXPERTS, BLK), lambda i: (0, i)),
        ],
        out_shape=[
            jax.ShapeDtypeStruct((TOP_K, N_TOKENS), jnp.int32),
            jax.ShapeDtypeStruct((TOP_K, N_TOKENS), jnp.float32),
            jax.ShapeDtypeStruct((1, 1), jnp.float32),
            jax.ShapeDtypeStruct((1, N_EXPERTS), jnp.float32),
            jax.ShapeDtypeStruct((N_EXPERTS, N_TOKENS), jnp.float32),
        ],
        scratch_shapes=[
            pltpu.VMEM((NBUF, BLK, HIDDEN), jnp.float32),
            pltpu.VMEM((N_EXPERTS, 2), jnp.float32),
            pltpu.SemaphoreType.DMA((NBUF,)),
        ],
    )(x, W)
    return (jnp.transpose(idx_t), jnp.transpose(tw_t), loss[0, 0],
            frac[0], jnp.transpose(probs_t))
